# baseline (speedup 1.0000x reference)
import functools

import jax
import jax.numpy as jnp
from jax import lax
from jax.experimental import pallas as pl
from jax.experimental.pallas import tpu as pltpu
from jax.experimental.pallas import tpu_sc as plsc

N = 75776
E = 1212416
G = 512
MN = 148
F0, F1, F2, F3 = 4, 16, 32, 64
OC = 8

NC, NS, L = 2, 16, 16
NW = NC * NS
f32 = jnp.float32
i32 = jnp.int32

CHUNK = 128
CPB = 8
BLK_E = CHUNK * CPB
EPT = E // NW
NBLK = EPT // BLK_E
EROWS_PT = EPT // CHUNK
NPT = N // NS
BROWS_PT = (N // CHUNK) // NS

NPAD = N + 16

NOPT = N // NW
NCH_FULL = NOPT // CHUNK
NCH_TAIL = NOPT - NCH_FULL * CHUNK
IBUF_LEN = (NCH_FULL + 1) * CHUNK

_MESH = plsc.VectorSubcoreMesh(
    core_axis_name="c", subcore_axis_name="s", num_cores=NC, num_subcores=NS
)
_SC_PARAMS = pltpu.CompilerParams(use_tc_tiling_on_sc=False)
_SC_PARAMS_NL = pltpu.CompilerParams(
    use_tc_tiling_on_sc=False, needs_layout_passes=False)


@functools.partial(
    pl.kernel,
    out_type=(
        jax.ShapeDtypeStruct((NC, N), f32),
        jax.ShapeDtypeStruct((G,), i32),
    ),
    mesh=_MESH,
    compiler_params=_SC_PARAMS,
    scratch_types=(
        pltpu.VMEM_SHARED((N,), f32),
        pltpu.VMEM_SHARED((G,), i32),
        pltpu.VMEM((CPB, CHUNK), i32),
        pltpu.VMEM((CHUNK,), f32),
        pltpu.VMEM((CHUNK,), i32),
        pltpu.VMEM((NPT,), f32),
        pltpu.VMEM((G,), i32),
    ),
)
def _deg_counts(dst2d, batch2d, zn_f, zg_i, deg_out, cnt_out,
                deg_sh, cnt_sh, idxb, ones_f, ones_i, dbuf, cbuf):
    c = lax.axis_index("c")
    s = lax.axis_index("s")
    for k in range(CHUNK // L):
        ones_f[pl.ds(k * L, L)] = jnp.ones((L,), f32)
        ones_i[pl.ds(k * L, L)] = jnp.ones((L,), i32)
    pltpu.sync_copy(zn_f, dbuf)
    pltpu.sync_copy(dbuf, deg_sh.at[pl.ds(s * NPT, NPT)])

    @pl.when(jnp.logical_and(c == 0, s == 0))
    def _():
        pltpu.sync_copy(zg_i, cbuf)
        pltpu.sync_copy(cbuf, cnt_sh)

    plsc.subcore_barrier()

    row0 = (c * NS + s) * EROWS_PT

    def blk(i, carry):
        pltpu.sync_copy(dst2d.at[pl.ds(row0 + i * CPB, CPB)], idxb)
        for j in range(CPB):
            pltpu.sync_copy(ones_f, deg_sh.at[idxb.at[j]], add=True)
        return carry

    lax.fori_loop(0, NBLK, blk, 0)

    @pl.when(c == 0)
    def _():
        def blk2(i, carry):
            pltpu.sync_copy(batch2d.at[pl.ds(s * BROWS_PT + i, 1)],
                            idxb.at[pl.ds(0, 1)])
            pltpu.sync_copy(ones_i, cnt_sh.at[idxb.at[0]], add=True)
            return carry

        lax.fori_loop(0, BROWS_PT, blk2, 0)

    plsc.subcore_barrier()
    pltpu.sync_copy(deg_sh.at[pl.ds(s * NPT, NPT)], dbuf)
    pltpu.sync_copy(dbuf, deg_out.at[c, pl.ds(s * NPT, NPT)])

    @pl.when(jnp.logical_and(c == 0, s == 0))
    def _():
        pltpu.sync_copy(cnt_sh, cbuf)
        pltpu.sync_copy(cbuf, cnt_out)


def _make_agg(C, nch):
    @functools.partial(
        pl.kernel,
        out_type=tuple(jax.ShapeDtypeStruct((NC, N, C), f32) for _ in range(nch)),
        mesh=_MESH,
        compiler_params=_SC_PARAMS,
        scratch_types=(
            pltpu.VMEM_SHARED((N, C), f32),
            pltpu.VMEM((2, CPB, CHUNK), i32),
            pltpu.VMEM((2, CPB, CHUNK), i32),
            pltpu.VMEM((2, BLK_E, C), f32),
            pltpu.VMEM((NPT, C), f32),
            pltpu.SemaphoreType.DMA,
            pltpu.SemaphoreType.DMA,
        ),
    )
    def agg(*refs):
        ys = refs[:nch]
        src2d, dst2d, zc_hbm = refs[nch:nch + 3]
        outs = refs[nch + 3:2 * nch + 3]
        acc, sidx, didx, gbuf, dbuf, gsem, ssem = refs[2 * nch + 3:]
        c = lax.axis_index("c")
        s = lax.axis_index("s")
        row0 = (c * NS + s) * EROWS_PT

        for ch in range(nch):
            y_hbm = ys[ch]
            out_hbm = outs[ch]

            def load_idx(b, blkno):
                r = row0 + blkno * CPB
                pltpu.sync_copy(src2d.at[pl.ds(r, CPB)], sidx.at[b])
                pltpu.sync_copy(dst2d.at[pl.ds(r, CPB)], didx.at[b])

            def gather(b, do_wait):
                for j in range(CPB):
                    d = pltpu.make_async_copy(
                        y_hbm.at[sidx.at[b, j]],
                        gbuf.at[b, pl.ds(j * CHUNK, CHUNK)], gsem)
                    d.wait() if do_wait else d.start()

            def scatter(b, do_wait):
                for j in range(CPB):
                    d = pltpu.make_async_copy(
                        gbuf.at[b, pl.ds(j * CHUNK, CHUNK)],
                        acc.at[didx.at[b, j]], ssem)
                    if do_wait:
                        d.wait()
                    else:
                        d.start(add=True)

            pltpu.sync_copy(zc_hbm, dbuf)
            pltpu.sync_copy(dbuf, acc.at[pl.ds(s * NPT, NPT)])
            plsc.subcore_barrier()

            load_idx(0, 0)
            gather(0, False)

            def blk(i, carry):
                b = lax.rem(i, 2)
                nb = 1 - b
                gather(b, True)
                scatter(b, False)
                load_idx(nb, i + 1)
                gather(nb, False)
                scatter(b, True)
                return carry

            lax.fori_loop(0, NBLK - 1, blk, 0)
            bl = (NBLK - 1) % 2
            gather(bl, True)
            scatter(bl, False)
            scatter(bl, True)

            plsc.subcore_barrier()
            pltpu.sync_copy(acc.at[pl.ds(s * NPT, NPT)], dbuf)
            pltpu.sync_copy(dbuf, out_hbm.at[c, pl.ds(s * NPT, NPT)])
            plsc.subcore_barrier()

    return agg


_agg8x1 = _make_agg(OC, 1)
_agg8x2 = _make_agg(OC, 2)
_agg8x4 = _make_agg(OC, 4)


@functools.partial(
    pl.kernel,
    out_type=jax.ShapeDtypeStruct((N, F3), f32),
    mesh=_MESH,
    compiler_params=pltpu.CompilerParams(
        use_tc_tiling_on_sc=False, needs_layout_passes=False),
    scratch_types=(
        pltpu.VMEM((G,), i32),
        pltpu.VMEM((G,), i32),
        pltpu.VMEM((G,), i32),
        pltpu.VMEM((NOPT,), i32),
        pltpu.VMEM((IBUF_LEN,), i32),
        pltpu.VMEM((CHUNK, F3), f32),
    ),
)
def _readout(h3_hbm, cnt_hbm, gi_hbm, out_hbm, cbuf, stab, etab, gibuf, ibuf, hbuf):
    c = lax.axis_index("c")
    s = lax.axis_index("s")
    w = c * NS + s
    base = w * NOPT
    pltpu.sync_copy(cnt_hbm, cbuf)

    def pbody(g, carry):
        ch = cbuf[pl.ds(g * L, L)]
        cum = plsc.cumsum(ch)
        etab[pl.ds(g * L, L)] = cum + carry
        stab[pl.ds(g * L, L)] = (cum + carry) - ch
        return carry + jnp.sum(ch)

    lax.fori_loop(0, G // L, pbody, jnp.asarray(0, i32))

    pltpu.sync_copy(gi_hbm.at[pl.ds(base, NOPT)], gibuf)

    def ibody(i, carry):
        gv = gibuf[pl.ds(i * L, L)]
        rv = (base + i * L) + lax.iota(i32, L)
        pv = rv - gv * MN
        sv = plsc.load_gather(stab, [gv])
        ev = plsc.load_gather(etab, [gv])
        idx = sv + pv
        ibuf[pl.ds(i * L, L)] = jnp.where(idx < ev, idx, N)
        return carry

    lax.fori_loop(0, NOPT // L, ibody, 0)
    for k in range(NOPT // L, IBUF_LEN // L):
        ibuf[pl.ds(k * L, L)] = jnp.full((L,), N, i32)

    for j in range(NCH_FULL + 1):
        nrows = CHUNK if j < NCH_FULL else NCH_TAIL
        pltpu.sync_copy(h3_hbm.at[ibuf.at[pl.ds(j * CHUNK, CHUNK)]], hbuf)
        pltpu.sync_copy(hbuf.at[pl.ds(0, nrows)],
                        out_hbm.at[pl.ds(base + j * CHUNK, nrows)])


GN = 592
NGRP = GN // L
NBLK_D = NOPT // GN
Q_RSQRT = 0x5F3759DF


def _rsqrt_newton(d):
    ii = jnp.int32(Q_RSQRT) - lax.shift_right_logical(plsc.bitcast(d, i32), 1)
    y = plsc.bitcast(ii, f32)
    for _ in range(3):
        y = y * (1.5 - 0.5 * d * y * y)
    return y


@functools.partial(
    pl.kernel,
    out_type=(
        jax.ShapeDtypeStruct((N,), f32),
        jax.ShapeDtypeStruct((N, OC), f32),
    ),
    mesh=_MESH,
    compiler_params=_SC_PARAMS_NL,
    scratch_types=(
        pltpu.VMEM((NOPT,), f32),
        pltpu.VMEM((NOPT,), f32),
        pltpu.VMEM((NOPT, F0), f32),
        pltpu.VMEM((NOPT,), f32),
        pltpu.VMEM((NOPT, OC), f32),
    ),
)
def _prep(deg_p, x_hbm, dis_out, y1_out, dA, dB, xb, disb, y1b):
    c = lax.axis_index("c")
    s = lax.axis_index("s")
    w = c * NS + s
    n0 = w * NOPT
    pltpu.sync_copy(deg_p.at[0, pl.ds(n0, NOPT)], dA)
    pltpu.sync_copy(deg_p.at[1, pl.ds(n0, NOPT)], dB)
    pltpu.sync_copy(x_hbm.at[pl.ds(n0, NOPT)], xb)

    def grp(g, carry):
        sl = pl.ds(g * L, L)
        dv = _rsqrt_newton(dA[sl] + dB[sl] + 1.0)
        disb[sl] = dv
        rows = g * L + lax.iota(i32, L)
        zv = jnp.zeros((L,), f32)
        for f in range(F0):
            cols = jnp.full((L,), f, i32)
            xv = plsc.load_gather(xb, [rows, cols])
            plsc.store_scatter(y1b, [rows, cols], xv * dv)
        for f in range(F0, OC):
            plsc.store_scatter(y1b, [rows, jnp.full((L,), f, i32)], zv)
        return carry

    lax.fori_loop(0, NOPT // L, grp, 0)
    pltpu.sync_copy(disb, dis_out.at[pl.ds(n0, NOPT)])
    pltpu.sync_copy(y1b, y1_out.at[pl.ds(n0, NOPT)])


def _make_dense(cin, cout, last):
    nci = cin // OC
    nco = cout // OC
    npass = 2 if cin * cout * 4 > 4096 else 1
    cpp = cin // npass
    if last:
        outs_t = (jax.ShapeDtypeStruct((NPAD, F3), f32),)
        obufs_t = (pltpu.VMEM((GN, F3), f32),)
    else:
        outs_t = tuple(jax.ShapeDtypeStruct((N, OC), f32) for _ in range(nco))
        obufs_t = tuple(pltpu.VMEM((GN, OC), f32) for _ in range(nco))

    @functools.partial(
        pl.kernel,
        out_type=outs_t,
        mesh=_MESH,
        compiler_params=_SC_PARAMS_NL,
        scratch_types=tuple(pltpu.VMEM((GN, OC), f32) for _ in range(3 * nci))
        + (pltpu.VMEM((GN,), f32),
           pltpu.SMEM((cpp, cout), f32),
           pltpu.SMEM((cout,), f32),
           pltpu.VMEM((cpp, cout), f32),
           pltpu.VMEM((cout,), f32))
        + obufs_t,
    )
    def dense(*refs):
        schunks = refs[:nci]
        ychunks = refs[nci:2 * nci]
        dis_h, w_h, b_h = refs[2 * nci:2 * nci + 3]
        pos = 2 * nci + 3
        outs = refs[pos:pos + (1 if last else nco)]
        pos += (1 if last else nco)
        tbufs = refs[pos:pos + 3 * nci]
        disb, wsm, bsm, wvm, bvm = refs[pos + 3 * nci:pos + 3 * nci + 5]
        obufs = refs[pos + 3 * nci + 5:]
        c = lax.axis_index("c")
        s = lax.axis_index("s")
        w = c * NS + s
        pltpu.sync_copy(b_h, bvm)
        for ok in range(cout // L):
            bv = bvm[pl.ds(ok * L, L)]
            for k in range(L):
                bsm[ok * L + k] = bv[k]

        if last:
            @pl.when(w == 0)
            def _():
                def zr(r, carry):
                    for k in range(F3 // L):
                        obufs[0][r, pl.ds(k * L, L)] = jnp.zeros((L,), f32)
                    return carry
                lax.fori_loop(0, NPAD - N, zr, 0)
                pltpu.sync_copy(obufs[0].at[pl.ds(0, NPAD - N)],
                                outs[0].at[pl.ds(N, NPAD - N)])

        def blk(nb, carry):
            n0 = w * NOPT + nb * GN
            for ch in range(nci):
                pltpu.sync_copy(schunks[ch].at[0, pl.ds(n0, GN)],
                                tbufs[3 * ch])
                pltpu.sync_copy(schunks[ch].at[1, pl.ds(n0, GN)],
                                tbufs[3 * ch + 1])
                pltpu.sync_copy(ychunks[ch].at[pl.ds(n0, GN)],
                                tbufs[3 * ch + 2])
            pltpu.sync_copy(dis_h.at[pl.ds(n0, GN)], disb)

            for p in range(npass):
                pltpu.sync_copy(w_h.at[pl.ds(p * cpp, cpp)], wvm)
                for i in range(cpp):
                    for ok in range(cout // L):
                        wrow = wvm[i, pl.ds(ok * L, L)]
                        for k in range(L):
                            wsm[i, ok * L + k] = wrow[k]

                def grp(g, carry2):
                    rows = g * L + lax.iota(i32, L)
                    dv = disb[pl.ds(g * L, L)]
                    tv = []
                    for ii in range(cpp):
                        i = p * cpp + ii
                        cols = jnp.full((L,), i % OC, i32)
                        ch = i // OC
                        tv.append(
                            plsc.load_gather(tbufs[3 * ch], [rows, cols])
                            + plsc.load_gather(tbufs[3 * ch + 1], [rows, cols])
                            + plsc.load_gather(tbufs[3 * ch + 2], [rows, cols]))
                    for o in range(cout):
                        acc = tv[0] * wsm[0, o]
                        for ii in range(1, cpp):
                            acc = acc + tv[ii] * wsm[ii, o]
                        ob = obufs[0] if last else obufs[o // OC]
                        oc = o if last else o % OC
                        ocols = jnp.full((L,), oc, i32)
                        if p < npass - 1:
                            plsc.store_scatter(ob, [rows, ocols], acc)
                        else:
                            if npass > 1:
                                acc = acc + plsc.load_gather(ob, [rows, ocols])
                            h = jnp.maximum(acc * dv + bsm[o], 0.0)
                            if not last:
                                h = h * dv
                            plsc.store_scatter(ob, [rows, ocols], h)
                    return carry2

                lax.fori_loop(0, NGRP, grp, 0)
            if last:
                pltpu.sync_copy(obufs[0], outs[0].at[pl.ds(n0, GN)])
            else:
                for k in range(nco):
                    pltpu.sync_copy(obufs[k], outs[k].at[pl.ds(n0, GN)])
            return carry

        lax.fori_loop(0, NBLK_D, blk, 0)

    return dense


_dense1 = _make_dense(OC, F1, last=False)
_dense2 = _make_dense(F1, F2, last=False)
_dense3 = _make_dense(F2, F3, last=True)


def kernel(x, edge_index, batch, W1, b1, W2, b2, W3, b3):
    src2d = edge_index[0].reshape(E // CHUNK, CHUNK)
    dst2d = edge_index[1].reshape(E // CHUNK, CHUNK)
    batch2d = batch.reshape(N // CHUNK, CHUNK)
    zn = jnp.zeros((NPT,), f32)
    zg = jnp.zeros((G,), i32)
    z8 = jnp.zeros((NPT, OC), f32)
    gi = jnp.arange(N, dtype=i32) // MN

    deg_p, counts = _deg_counts(dst2d, batch2d, zn, zg)
    dis, y1 = _prep(deg_p, x)

    (s1,) = _agg8x1(y1, src2d, dst2d, z8)
    W1p = jnp.concatenate([W1, jnp.zeros((OC - F0, F1), f32)], axis=0)
    y2a, y2b = _dense1(s1, y1, dis, W1p, b1)

    s2a, s2b = _agg8x2(y2a, y2b, src2d, dst2d, z8)
    y3a, y3b, y3c, y3d = _dense2(s2a, s2b, y2a, y2b, dis, W2, b2)

    s3 = _agg8x4(y3a, y3b, y3c, y3d, src2d, dst2d, z8)
    (h3,) = _dense3(s3[0], s3[1], s3[2], s3[3], y3a, y3b, y3c, y3d,
                    dis, W3, b3)

    dense = _readout(h3, counts, gi)
    return dense.reshape(G, MN * F3)


# speedup vs baseline: 1.2637x; 1.2637x over previous
import functools

import jax
import jax.numpy as jnp
from jax import lax
from jax.experimental import pallas as pl
from jax.experimental.pallas import tpu as pltpu
from jax.experimental.pallas import tpu_sc as plsc

N = 75776
E = 1212416
G = 512
MN = 148
F0, F1, F2, F3 = 4, 16, 32, 64
OC = 8

NC, NS, L = 2, 16, 16
NW = NC * NS
f32 = jnp.float32
i32 = jnp.int32

CHUNK = 128
CPB = 8
BLK_E = CHUNK * CPB
EPT = E // NW
NBLK = EPT // BLK_E
EROWS_PT = EPT // CHUNK
NPT = N // NS
BROWS_PT = (N // CHUNK) // NS

NPAD = N + 16

NOPT = N // NW
NCH_FULL = NOPT // CHUNK
NCH_TAIL = NOPT - NCH_FULL * CHUNK
IBUF_LEN = (NCH_FULL + 1) * CHUNK

_MESH = plsc.VectorSubcoreMesh(
    core_axis_name="c", subcore_axis_name="s", num_cores=NC, num_subcores=NS
)
_SC_PARAMS = pltpu.CompilerParams(use_tc_tiling_on_sc=False)
_SC_PARAMS_NL = pltpu.CompilerParams(
    use_tc_tiling_on_sc=False, needs_layout_passes=False)


@functools.partial(
    pl.kernel,
    out_type=(
        jax.ShapeDtypeStruct((NC, N), f32),
        jax.ShapeDtypeStruct((G,), i32),
    ),
    mesh=_MESH,
    compiler_params=_SC_PARAMS,
    scratch_types=(
        pltpu.VMEM_SHARED((N,), f32),
        pltpu.VMEM_SHARED((G,), i32),
        pltpu.VMEM((CPB, CHUNK), i32),
        pltpu.VMEM((CHUNK,), f32),
        pltpu.VMEM((CHUNK,), i32),
        pltpu.VMEM((NPT,), f32),
        pltpu.VMEM((G,), i32),
    ),
)
def _deg_counts(dst2d, batch2d, zn_f, zg_i, deg_out, cnt_out,
                deg_sh, cnt_sh, idxb, ones_f, ones_i, dbuf, cbuf):
    c = lax.axis_index("c")
    s = lax.axis_index("s")
    for k in range(CHUNK // L):
        ones_f[pl.ds(k * L, L)] = jnp.ones((L,), f32)
        ones_i[pl.ds(k * L, L)] = jnp.ones((L,), i32)
    pltpu.sync_copy(zn_f, dbuf)
    pltpu.sync_copy(dbuf, deg_sh.at[pl.ds(s * NPT, NPT)])

    @pl.when(jnp.logical_and(c == 0, s == 0))
    def _():
        pltpu.sync_copy(zg_i, cbuf)
        pltpu.sync_copy(cbuf, cnt_sh)

    plsc.subcore_barrier()

    row0 = (c * NS + s) * EROWS_PT

    def blk(i, carry):
        pltpu.sync_copy(dst2d.at[pl.ds(row0 + i * CPB, CPB)], idxb)
        for j in range(CPB):
            pltpu.sync_copy(ones_f, deg_sh.at[idxb.at[j]], add=True)
        return carry

    lax.fori_loop(0, NBLK, blk, 0)

    @pl.when(c == 0)
    def _():
        def blk2(i, carry):
            pltpu.sync_copy(batch2d.at[pl.ds(s * BROWS_PT + i, 1)],
                            idxb.at[pl.ds(0, 1)])
            pltpu.sync_copy(ones_i, cnt_sh.at[idxb.at[0]], add=True)
            return carry

        lax.fori_loop(0, BROWS_PT, blk2, 0)

    plsc.subcore_barrier()
    pltpu.sync_copy(deg_sh.at[pl.ds(s * NPT, NPT)], dbuf)
    pltpu.sync_copy(dbuf, deg_out.at[c, pl.ds(s * NPT, NPT)])

    @pl.when(jnp.logical_and(c == 0, s == 0))
    def _():
        pltpu.sync_copy(cnt_sh, cbuf)
        pltpu.sync_copy(cbuf, cnt_out)


def _make_agg(C, nch):
    @functools.partial(
        pl.kernel,
        out_type=tuple(jax.ShapeDtypeStruct((NC, N, C), f32) for _ in range(nch)),
        mesh=_MESH,
        compiler_params=_SC_PARAMS,
        scratch_types=(
            pltpu.VMEM_SHARED((N, C), f32),
            pltpu.VMEM((2, CPB, CHUNK), i32),
            pltpu.VMEM((2, CPB, CHUNK), i32),
            pltpu.VMEM((2, BLK_E, C), f32),
            pltpu.VMEM((NPT, C), f32),
            pltpu.SemaphoreType.DMA,
            pltpu.SemaphoreType.DMA,
        ),
    )
    def agg(*refs):
        ys = refs[:nch]
        src2d, dst2d, zc_hbm = refs[nch:nch + 3]
        outs = refs[nch + 3:2 * nch + 3]
        acc, sidx, didx, gbuf, dbuf, gsem, ssem = refs[2 * nch + 3:]
        c = lax.axis_index("c")
        s = lax.axis_index("s")
        row0 = (c * NS + s) * EROWS_PT

        for ch in range(nch):
            y_hbm = ys[ch]
            out_hbm = outs[ch]

            def load_idx(b, blkno):
                r = row0 + blkno * CPB
                pltpu.sync_copy(src2d.at[pl.ds(r, CPB)], sidx.at[b])
                pltpu.sync_copy(dst2d.at[pl.ds(r, CPB)], didx.at[b])

            def gather(b, do_wait):
                for j in range(CPB):
                    d = pltpu.make_async_copy(
                        y_hbm.at[sidx.at[b, j]],
                        gbuf.at[b, pl.ds(j * CHUNK, CHUNK)], gsem)
                    d.wait() if do_wait else d.start()

            def scatter(b, do_wait):
                for j in range(CPB):
                    d = pltpu.make_async_copy(
                        gbuf.at[b, pl.ds(j * CHUNK, CHUNK)],
                        acc.at[didx.at[b, j]], ssem)
                    if do_wait:
                        d.wait()
                    else:
                        d.start(add=True)

            pltpu.sync_copy(zc_hbm, dbuf)
            pltpu.sync_copy(dbuf, acc.at[pl.ds(s * NPT, NPT)])
            plsc.subcore_barrier()

            load_idx(0, 0)
            gather(0, False)

            def blk(i, carry):
                b = lax.rem(i, 2)
                nb = 1 - b
                gather(b, True)
                scatter(b, False)
                load_idx(nb, i + 1)
                gather(nb, False)
                scatter(b, True)
                return carry

            lax.fori_loop(0, NBLK - 1, blk, 0)
            bl = (NBLK - 1) % 2
            gather(bl, True)
            scatter(bl, False)
            scatter(bl, True)

            plsc.subcore_barrier()
            pltpu.sync_copy(acc.at[pl.ds(s * NPT, NPT)], dbuf)
            pltpu.sync_copy(dbuf, out_hbm.at[c, pl.ds(s * NPT, NPT)])
            plsc.subcore_barrier()

    return agg


_agg8x1 = _make_agg(OC, 1)
_agg8x2 = _make_agg(OC, 2)
_agg8x4 = _make_agg(OC, 4)


@functools.partial(
    pl.kernel,
    out_type=jax.ShapeDtypeStruct((N, F3), f32),
    mesh=_MESH,
    compiler_params=pltpu.CompilerParams(
        use_tc_tiling_on_sc=False, needs_layout_passes=False),
    scratch_types=(
        pltpu.VMEM((G,), i32),
        pltpu.VMEM((G,), i32),
        pltpu.VMEM((G,), i32),
        pltpu.VMEM((NOPT,), i32),
        pltpu.VMEM((IBUF_LEN,), i32),
        pltpu.VMEM((CHUNK, F3), f32),
    ),
)
def _readout(h3_hbm, cnt_hbm, gi_hbm, out_hbm, cbuf, stab, etab, gibuf, ibuf, hbuf):
    c = lax.axis_index("c")
    s = lax.axis_index("s")
    w = c * NS + s
    base = w * NOPT
    pltpu.sync_copy(cnt_hbm, cbuf)

    def pbody(g, carry):
        ch = cbuf[pl.ds(g * L, L)]
        cum = plsc.cumsum(ch)
        etab[pl.ds(g * L, L)] = cum + carry
        stab[pl.ds(g * L, L)] = (cum + carry) - ch
        return carry + jnp.sum(ch)

    lax.fori_loop(0, G // L, pbody, jnp.asarray(0, i32))

    pltpu.sync_copy(gi_hbm.at[pl.ds(base, NOPT)], gibuf)

    def ibody(i, carry):
        gv = gibuf[pl.ds(i * L, L)]
        rv = (base + i * L) + lax.iota(i32, L)
        pv = rv - gv * MN
        sv = plsc.load_gather(stab, [gv])
        ev = plsc.load_gather(etab, [gv])
        idx = sv + pv
        ibuf[pl.ds(i * L, L)] = jnp.where(idx < ev, idx, N)
        return carry

    lax.fori_loop(0, NOPT // L, ibody, 0)
    for k in range(NOPT // L, IBUF_LEN // L):
        ibuf[pl.ds(k * L, L)] = jnp.full((L,), N, i32)

    for j in range(NCH_FULL + 1):
        nrows = CHUNK if j < NCH_FULL else NCH_TAIL
        pltpu.sync_copy(h3_hbm.at[ibuf.at[pl.ds(j * CHUNK, CHUNK)]], hbuf)
        pltpu.sync_copy(hbuf.at[pl.ds(0, nrows)],
                        out_hbm.at[pl.ds(base + j * CHUNK, nrows)])


GN = 592
NGRP = GN // L
NBLK_D = NOPT // GN
Q_RSQRT = 0x5F3759DF


def _rsqrt_newton(d):
    ii = jnp.int32(Q_RSQRT) - lax.shift_right_logical(plsc.bitcast(d, i32), 1)
    y = plsc.bitcast(ii, f32)
    for _ in range(3):
        y = y * (1.5 - 0.5 * d * y * y)
    return y


@functools.partial(
    pl.kernel,
    out_type=(
        jax.ShapeDtypeStruct((N,), f32),
        jax.ShapeDtypeStruct((N, OC), f32),
    ),
    mesh=_MESH,
    compiler_params=_SC_PARAMS_NL,
    scratch_types=(
        pltpu.VMEM((NOPT,), f32),
        pltpu.VMEM((NOPT,), f32),
        pltpu.VMEM((NOPT, F0), f32),
        pltpu.VMEM((NOPT,), f32),
        pltpu.VMEM((NOPT, OC), f32),
    ),
)
def _prep(deg_p, x_hbm, dis_out, y1_out, dA, dB, xb, disb, y1b):
    c = lax.axis_index("c")
    s = lax.axis_index("s")
    w = c * NS + s
    n0 = w * NOPT
    pltpu.sync_copy(deg_p.at[0, pl.ds(n0, NOPT)], dA)
    pltpu.sync_copy(deg_p.at[1, pl.ds(n0, NOPT)], dB)
    pltpu.sync_copy(x_hbm.at[pl.ds(n0, NOPT)], xb)

    def grp(g, carry):
        sl = pl.ds(g * L, L)
        dv = _rsqrt_newton(dA[sl] + dB[sl] + 1.0)
        disb[sl] = dv
        rows = g * L + lax.iota(i32, L)
        zv = jnp.zeros((L,), f32)
        for f in range(F0):
            cols = jnp.full((L,), f, i32)
            xv = plsc.load_gather(xb, [rows, cols])
            plsc.store_scatter(y1b, [rows, cols], xv * dv)
        for f in range(F0, OC):
            plsc.store_scatter(y1b, [rows, jnp.full((L,), f, i32)], zv)
        return carry

    lax.fori_loop(0, NOPT // L, grp, 0)
    pltpu.sync_copy(disb, dis_out.at[pl.ds(n0, NOPT)])
    pltpu.sync_copy(y1b, y1_out.at[pl.ds(n0, NOPT)])


def _make_dense(cin, cout, last):
    nci = cin // OC
    nco = cout // OC
    if last:
        outs_t = (jax.ShapeDtypeStruct((NPAD, F3), f32),)
        obufs_t = (pltpu.VMEM((GN, F3), f32),)
    else:
        outs_t = tuple(jax.ShapeDtypeStruct((N, OC), f32) for _ in range(nco))
        obufs_t = tuple(pltpu.VMEM((GN, OC), f32) for _ in range(nco))

    @functools.partial(
        pl.kernel,
        out_type=outs_t,
        mesh=_MESH,
        compiler_params=_SC_PARAMS_NL,
        scratch_types=tuple(pltpu.VMEM((GN, OC), f32) for _ in range(3 * nci))
        + (pltpu.VMEM((GN,), f32),
           pltpu.VMEM((cin, cout), f32),
           pltpu.VMEM((cout,), f32))
        + obufs_t,
    )
    def dense(*refs):
        schunks = refs[:nci]
        ychunks = refs[nci:2 * nci]
        dis_h, w_h, b_h = refs[2 * nci:2 * nci + 3]
        pos = 2 * nci + 3
        outs = refs[pos:pos + (1 if last else nco)]
        pos += (1 if last else nco)
        tbufs = refs[pos:pos + 3 * nci]
        disb, wb, bb = refs[pos + 3 * nci:pos + 3 * nci + 3]
        obufs = refs[pos + 3 * nci + 3:]
        c = lax.axis_index("c")
        s = lax.axis_index("s")
        w = c * NS + s
        pltpu.sync_copy(w_h, wb)
        pltpu.sync_copy(b_h, bb)

        if last:
            @pl.when(w == 0)
            def _():
                def zr(r, carry):
                    for k in range(F3 // L):
                        obufs[0][r, pl.ds(k * L, L)] = jnp.zeros((L,), f32)
                    return carry
                lax.fori_loop(0, NPAD - N, zr, 0)
                pltpu.sync_copy(obufs[0].at[pl.ds(0, NPAD - N)],
                                outs[0].at[pl.ds(N, NPAD - N)])

        def blk(nb, carry):
            n0 = w * NOPT + nb * GN
            for ch in range(nci):
                pltpu.sync_copy(schunks[ch].at[0, pl.ds(n0, GN)],
                                tbufs[3 * ch])
                pltpu.sync_copy(schunks[ch].at[1, pl.ds(n0, GN)],
                                tbufs[3 * ch + 1])
                pltpu.sync_copy(ychunks[ch].at[pl.ds(n0, GN)],
                                tbufs[3 * ch + 2])
            pltpu.sync_copy(dis_h.at[pl.ds(n0, GN)], disb)

            def grp(g, carry2):
                rows = g * L + lax.iota(i32, L)
                dv = disb[pl.ds(g * L, L)]
                tv = []
                for i in range(cin):
                    cols = jnp.full((L,), i % OC, i32)
                    ch = i // OC
                    tv.append(plsc.load_gather(tbufs[3 * ch], [rows, cols])
                              + plsc.load_gather(tbufs[3 * ch + 1], [rows, cols])
                              + plsc.load_gather(tbufs[3 * ch + 2], [rows, cols]))
                for obk in range(cout // L):
                    osl = pl.ds(obk * L, L)
                    wvs = [wb[i, osl] for i in range(cin)]
                    bv = bb[osl]
                    for k in range(L):
                        o = obk * L + k
                        acc = tv[0] * wvs[0][k]
                        for i in range(1, cin):
                            acc = acc + tv[i] * wvs[i][k]
                        h = jnp.maximum(acc * dv + bv[k], 0.0)
                        if not last:
                            h = h * dv
                        ob = obufs[0] if last else obufs[o // OC]
                        oc = o if last else o % OC
                        plsc.store_scatter(
                            ob, [rows, jnp.full((L,), oc, i32)], h)
                return carry2

            lax.fori_loop(0, NGRP, grp, 0)
            if last:
                pltpu.sync_copy(obufs[0], outs[0].at[pl.ds(n0, GN)])
            else:
                for k in range(nco):
                    pltpu.sync_copy(obufs[k], outs[k].at[pl.ds(n0, GN)])
            return carry

        lax.fori_loop(0, NBLK_D, blk, 0)

    return dense


_dense1 = _make_dense(OC, F1, last=False)
_dense2 = _make_dense(F1, F2, last=False)
_dense3 = _make_dense(F2, F3, last=True)


def kernel(x, edge_index, batch, W1, b1, W2, b2, W3, b3):
    src2d = edge_index[0].reshape(E // CHUNK, CHUNK)
    dst2d = edge_index[1].reshape(E // CHUNK, CHUNK)
    batch2d = batch.reshape(N // CHUNK, CHUNK)
    zn = jnp.zeros((NPT,), f32)
    zg = jnp.zeros((G,), i32)
    z8 = jnp.zeros((NPT, OC), f32)
    gi = jnp.arange(N, dtype=i32) // MN

    deg_p, counts = _deg_counts(dst2d, batch2d, zn, zg)
    dis, y1 = _prep(deg_p, x)

    (s1,) = _agg8x1(y1, src2d, dst2d, z8)
    W1p = jnp.concatenate([W1, jnp.zeros((OC - F0, F1), f32)], axis=0)
    y2a, y2b = _dense1(s1, y1, dis, W1p, b1)

    s2a, s2b = _agg8x2(y2a, y2b, src2d, dst2d, z8)
    y3a, y3b, y3c, y3d = _dense2(s2a, s2b, y2a, y2b, dis, W2, b2)

    s3 = _agg8x4(y3a, y3b, y3c, y3d, src2d, dst2d, z8)
    (h3,) = _dense3(s3[0], s3[1], s3[2], s3[3], y3a, y3b, y3c, y3d,
                    dis, W3, b3)

    dense = _readout(h3, counts, gi)
    return dense.reshape(G, MN * F3)
